# baseline (speedup 1.0000x reference)
import functools

import jax
import jax.numpy as jnp
from jax import lax
from jax.experimental import pallas as pl
from jax.experimental.pallas import tpu as pltpu
from jax.experimental.pallas import tpu_sc as plsc

_NC = 2
_NS = 16
_NW = _NC * _NS
_CH = 80


def _htab_call(x, W, b_type, *, n_blk):
    n, d = x.shape
    t = b_type.shape[0]

    def body(x_ref, w_ref, b_ref, out_ref):
        h = lax.dot(
            x_ref[...],
            w_ref[...],
            precision=lax.Precision.HIGHEST,
            preferred_element_type=jnp.float32,
        )
        out_ref[...] = (h[:, None, :] + b_ref[...][None, :, :]).reshape(
            n_blk * t, d
        )

    return pl.pallas_call(
        body,
        grid=(n // n_blk,),
        in_specs=[
            pl.BlockSpec((n_blk, d), lambda i: (i, 0)),
            pl.BlockSpec((d, d), lambda i: (0, 0)),
            pl.BlockSpec((t, d), lambda i: (0, 0)),
        ],
        out_specs=pl.BlockSpec((n_blk * t, d), lambda i: (i, 0)),
        out_shape=jax.ShapeDtypeStruct((n * t, d), jnp.float32),
    )(x, W, b_type)


def _final_call(partials, x, W_self, *, n_blk):
    n, d = x.shape

    def body(p_ref, x_ref, w_ref, out_ref):
        s = lax.dot(
            x_ref[...],
            w_ref[...],
            precision=lax.Precision.HIGHEST,
            preferred_element_type=jnp.float32,
        )
        out_ref[...] = jnp.maximum(p_ref[0] + p_ref[1] + s, 0.0)

    return pl.pallas_call(
        body,
        grid=(n // n_blk,),
        in_specs=[
            pl.BlockSpec((2, n_blk, d), lambda i: (0, i, 0)),
            pl.BlockSpec((n_blk, d), lambda i: (i, 0)),
            pl.BlockSpec((d, d), lambda i: (0, 0)),
        ],
        out_specs=pl.BlockSpec((n_blk, d), lambda i: (i, 0)),
        out_shape=jax.ShapeDtypeStruct((n, d), jnp.float32),
    )(partials, x, W_self)


def _sc_aggregate(htab2, gi1, di1, *, n, n_acc, d, epw, n_ch):
    ch = _CH
    npt = (n // _NS) // 8 * 8
    wrem = n - _NS * npt
    zrem = n_acc - _NS * npt
    zch = 48
    assert npt % zch == 0 and wrem % 8 == 0 and zrem % 8 == 0
    assert max(wrem, zrem) <= zch
    assert n_ch * ch == epw and ch % 8 == 0 and n_ch >= 8
    mesh = plsc.VectorSubcoreMesh(
        core_axis_name="c", subcore_axis_name="s", num_cores=_NC, num_subcores=_NS
    )

    @functools.partial(
        pl.kernel,
        mesh=mesh,
        out_type=jax.ShapeDtypeStruct((_NC, n, d), jnp.float32),
        scratch_types=[
            pltpu.VMEM((4, ch), jnp.int32),
            pltpu.VMEM((4, ch), jnp.int32),
            pltpu.VMEM((4, ch, d), jnp.float32),
            pltpu.VMEM((zch, d), jnp.float32),
            pltpu.VMEM_SHARED((n_acc, d), jnp.float32),
            pltpu.SemaphoreType.DMA((4,)),
            pltpu.SemaphoreType.DMA((4,)),
            pltpu.SemaphoreType.DMA((4,)),
        ],
    )
    def run(tab_hbm, gi_hbm, di_hbm, out_hbm,
            gi_v, di_v, rows_v, zero_v, acc_sh, gsems, dsems, rsems):
        cid = lax.axis_index("c")
        sid = lax.axis_index("s")
        wid = cid * _NS + sid

        def zero_row(i, carry):
            for c in range(d // 16):
                zero_v[i, pl.ds(c * 16, 16)] = jnp.zeros((16,), jnp.float32)
            return carry

        lax.fori_loop(0, zch, zero_row, 0)
        for k in range(npt // zch):
            pltpu.sync_copy(zero_v, acc_sh.at[pl.ds(sid * npt + k * zch, zch)])

        @pl.when(sid == _NS - 1)
        def _zero_tail():
            pltpu.sync_copy(
                zero_v.at[pl.ds(0, zrem)], acc_sh.at[pl.ds(_NS * npt, zrem)]
            )

        plsc.subcore_barrier()

        base_e = wid * epw

        def start_idx(c):
            off = base_e + c * ch
            s = c % 4
            pltpu.async_copy(gi_hbm.at[pl.ds(off, ch)], gi_v.at[s], gsems.at[s])
            pltpu.async_copy(di_hbm.at[pl.ds(off, ch)], di_v.at[s], dsems.at[s])

        def wait_idx(c):
            off = base_e + c * ch
            s = c % 4
            pltpu.make_async_copy(
                gi_hbm.at[pl.ds(off, ch)], gi_v.at[s], gsems.at[s]
            ).wait()
            pltpu.make_async_copy(
                di_hbm.at[pl.ds(off, ch)], di_v.at[s], dsems.at[s]
            ).wait()

        def start_gather(c):
            pltpu.async_copy(
                tab_hbm.at[gi_v.at[c % 4]], rows_v.at[c % 4], rsems.at[c % 4]
            )

        def wait_gather(c):
            pltpu.make_async_copy(
                tab_hbm.at[gi_v.at[c % 4]], rows_v.at[c % 4], rsems.at[c % 4]
            ).wait()

        def scatter(c):
            pltpu.sync_copy(
                rows_v.at[c % 4], acc_sh.at[di_v.at[c % 4]], add=True
            )

        for c in range(3):
            start_idx(c)
        for c in range(3):
            wait_idx(c)
            start_gather(c)
        start_idx(3)
        for c in range(n_ch):
            if c + 3 < n_ch:
                wait_idx(c + 3)
                start_gather(c + 3)
            wait_gather(c)
            scatter(c)
            if c + 4 < n_ch:
                start_idx(c + 4)

        plsc.subcore_barrier()

        pltpu.sync_copy(
            acc_sh.at[pl.ds(sid * npt, npt)],
            out_hbm.at[cid, pl.ds(sid * npt, npt)],
        )

        @pl.when(sid == _NS - 1)
        def _write_tail():
            pltpu.sync_copy(
                acc_sh.at[pl.ds(_NS * npt, wrem)],
                out_hbm.at[cid, pl.ds(_NS * npt, wrem)],
            )

    return run(htab2, gi1, di1)


def kernel(x, edge_index, edge_type, W, W_self, b_type):
    n, d = x.shape
    e = edge_index.shape[1]
    t = b_type.shape[0]
    assert e % (_NW * _CH) == 0
    epw = e // _NW
    n_ch = epw // _CH
    n_acc = -(-n // 8) * 8

    src = edge_index[0]
    dst = edge_index[1]
    gidx = src * t + edge_type

    htab = _htab_call(x, W, b_type, n_blk=2000)
    partials = _sc_aggregate(
        htab, gidx, dst,
        n=n, n_acc=n_acc, d=d, epw=epw, n_ch=n_ch,
    )
    return _final_call(partials, x, W_self, n_blk=1000)


# speedup vs baseline: 1.1399x; 1.1399x over previous
import functools

import jax
import jax.numpy as jnp
from jax import lax
from jax.experimental import pallas as pl
from jax.experimental.pallas import tpu as pltpu
from jax.experimental.pallas import tpu_sc as plsc

_NC = 2
_NS = 16
_NW = _NC * _NS
_CH = 125


def _htab_call(x, W, b_type, *, n_blk):
    n, d = x.shape
    t = b_type.shape[0]

    def body(x_ref, w_ref, b_ref, out_ref):
        h = lax.dot(
            x_ref[...],
            w_ref[...],
            precision=lax.Precision.HIGHEST,
            preferred_element_type=jnp.float32,
        )
        out_ref[...] = (h[:, None, :] + b_ref[...][None, :, :]).reshape(
            n_blk * t, d
        )

    return pl.pallas_call(
        body,
        grid=(n // n_blk,),
        in_specs=[
            pl.BlockSpec((n_blk, d), lambda i: (i, 0)),
            pl.BlockSpec((d, d), lambda i: (0, 0)),
            pl.BlockSpec((t, d), lambda i: (0, 0)),
        ],
        out_specs=pl.BlockSpec((n_blk * t, d), lambda i: (i, 0)),
        out_shape=jax.ShapeDtypeStruct((n * t, d), jnp.float32),
    )(x, W, b_type)


def _final_call(partials, x, W_self, *, n_blk):
    n, d = x.shape

    def body(p_ref, x_ref, w_ref, out_ref):
        s = lax.dot(
            x_ref[...],
            w_ref[...],
            precision=lax.Precision.HIGHEST,
            preferred_element_type=jnp.float32,
        )
        out_ref[...] = jnp.maximum(p_ref[0] + p_ref[1] + s, 0.0)

    return pl.pallas_call(
        body,
        grid=(n // n_blk,),
        in_specs=[
            pl.BlockSpec((2, n_blk, d), lambda i: (0, i, 0)),
            pl.BlockSpec((n_blk, d), lambda i: (i, 0)),
            pl.BlockSpec((d, d), lambda i: (0, 0)),
        ],
        out_specs=pl.BlockSpec((n_blk, d), lambda i: (i, 0)),
        out_shape=jax.ShapeDtypeStruct((n, d), jnp.float32),
    )(partials, x, W_self)


def _sc_aggregate(htab2, gi4, di4, *, n, n_acc, d, n_ch):
    ch = _CH
    npt = (n // _NS) // 8 * 8
    wrem = n - _NS * npt
    zrem = n_acc - _NS * npt
    zch = 104
    assert npt % zch == 0 and wrem % 8 == 0 and zrem % 8 == 0
    assert max(wrem, zrem) <= ch and zch <= ch
    assert n_ch % 4 == 0 and n_ch >= 8
    mesh = plsc.VectorSubcoreMesh(
        core_axis_name="c", subcore_axis_name="s", num_cores=_NC, num_subcores=_NS
    )

    @functools.partial(
        pl.kernel,
        mesh=mesh,
        out_type=jax.ShapeDtypeStruct((_NC, n, d), jnp.float32),
        scratch_types=[
            pltpu.VMEM((4, 1, ch), jnp.int32),
            pltpu.VMEM((4, 1, ch), jnp.int32),
            pltpu.VMEM((2, ch, d), jnp.float32),
            pltpu.VMEM((zch, d), jnp.float32),
            pltpu.VMEM_SHARED((n_acc, d), jnp.float32),
            pltpu.SemaphoreType.DMA((4,)),
            pltpu.SemaphoreType.DMA((4,)),
            pltpu.SemaphoreType.DMA((2,)),
        ],
    )
    def run(tab_hbm, gi_hbm, di_hbm, out_hbm,
            gi_v, di_v, rows_v, zero_v, acc_sh, gsems, dsems, rsems):
        cid = lax.axis_index("c")
        sid = lax.axis_index("s")
        wid = cid * _NS + sid

        def zero_row(i, carry):
            for c in range(d // 16):
                zero_v[i, pl.ds(c * 16, 16)] = jnp.zeros((16,), jnp.float32)
            return carry

        lax.fori_loop(0, zch, zero_row, 0)
        for k in range(npt // zch):
            pltpu.sync_copy(zero_v, acc_sh.at[pl.ds(sid * npt + k * zch, zch)])

        @pl.when(sid == _NS - 1)
        def _zero_tail():
            pltpu.sync_copy(
                zero_v.at[pl.ds(0, zrem)], acc_sh.at[pl.ds(_NS * npt, zrem)]
            )

        plsc.subcore_barrier()

        def start_idx(j, s):
            pltpu.async_copy(gi_hbm.at[wid, j], gi_v.at[s], gsems.at[s])
            pltpu.async_copy(di_hbm.at[wid, j], di_v.at[s], dsems.at[s])

        def wait_idx(j, s):
            pltpu.make_async_copy(
                gi_hbm.at[wid, j], gi_v.at[s], gsems.at[s]
            ).wait()
            pltpu.make_async_copy(
                di_hbm.at[wid, j], di_v.at[s], dsems.at[s]
            ).wait()

        def start_gather(s, r):
            pltpu.async_copy(
                tab_hbm.at[gi_v.at[s, 0]], rows_v.at[r], rsems.at[r]
            )

        def wait_gather(s, r):
            pltpu.make_async_copy(
                tab_hbm.at[gi_v.at[s, 0]], rows_v.at[r], rsems.at[r]
            ).wait()

        def scatter(s, r):
            pltpu.sync_copy(
                rows_v.at[r], acc_sh.at[di_v.at[s, 0]], add=True
            )

        def step(j, jj):
            wait_idx(jj + 1, (j + 1) % 4)
            start_gather((j + 1) % 4, (j + 1) % 2)
            wait_gather(j % 4, j % 2)
            scatter(j % 4, j % 2)
            start_idx(jj + 3, (j + 3) % 4)

        start_idx(0, 0)
        start_idx(1, 1)
        start_idx(2, 2)
        wait_idx(0, 0)
        start_gather(0, 0)

        def quad(k, carry):
            j4 = 4 * k
            for u in range(4):
                step(u, j4 + u)
            return carry

        lax.fori_loop(0, (n_ch - 4) // 4, quad, 0)

        base = n_ch - 4
        for u in range(4):
            j = base + u
            if u < 3:
                wait_idx(j + 1, (j + 1) % 4)
                start_gather((j + 1) % 4, (j + 1) % 2)
            wait_gather(j % 4, j % 2)
            scatter(j % 4, j % 2)
            if u == 0:
                start_idx(n_ch - 1, (n_ch - 1) % 4)

        plsc.subcore_barrier()

        pltpu.sync_copy(
            acc_sh.at[pl.ds(sid * npt, npt)],
            out_hbm.at[cid, pl.ds(sid * npt, npt)],
        )

        @pl.when(sid == _NS - 1)
        def _write_tail():
            pltpu.sync_copy(
                acc_sh.at[pl.ds(_NS * npt, wrem)],
                out_hbm.at[cid, pl.ds(_NS * npt, wrem)],
            )

    return run(htab2, gi4, di4)


def kernel(x, edge_index, edge_type, W, W_self, b_type):
    n, d = x.shape
    e = edge_index.shape[1]
    t = b_type.shape[0]
    assert e % _NW == 0
    epw = e // _NW
    n_ch = -(-epw // _CH)
    pad = n_ch * _CH - epw
    n_acc = -(-(n + 1) // 8) * 8

    src = edge_index[0]
    dst = edge_index[1]
    gidx = src * t + edge_type

    def chunked(idx, fill):
        idx = idx.reshape(_NW, epw)
        if pad:
            filler = jnp.full((_NW, pad), fill, dtype=jnp.int32)
            idx = jnp.concatenate([idx, filler], axis=1)
        return idx.reshape(_NW, n_ch, 1, _CH)

    htab = _htab_call(x, W, b_type, n_blk=2000)
    partials = _sc_aggregate(
        htab,
        chunked(gidx, 0),
        chunked(dst, n),
        n=n, n_acc=n_acc, d=d, n_ch=n_ch,
    )
    return _final_call(partials, x, W_self, n_blk=1000)


# speedup vs baseline: 1.1652x; 1.0222x over previous
import functools

import jax
import jax.numpy as jnp
from jax import lax
from jax.experimental import pallas as pl
from jax.experimental.pallas import tpu as pltpu
from jax.experimental.pallas import tpu_sc as plsc

_NC = 2
_NS = 16
_NW = _NC * _NS
_CH = 125


def _htab_call(x, W, b_type, *, n_blk):
    n, d = x.shape
    t = b_type.shape[0]

    def body(x_ref, w_ref, b_ref, out_ref):
        h = lax.dot(
            x_ref[...],
            w_ref[...],
            precision=lax.Precision.HIGHEST,
            preferred_element_type=jnp.float32,
        )
        out_ref[...] = (h[:, None, :] + b_ref[...][None, :, :]).reshape(
            n_blk * t, d
        )

    return pl.pallas_call(
        body,
        grid=(n // n_blk,),
        in_specs=[
            pl.BlockSpec((n_blk, d), lambda i: (i, 0)),
            pl.BlockSpec((d, d), lambda i: (0, 0)),
            pl.BlockSpec((t, d), lambda i: (0, 0)),
        ],
        out_specs=pl.BlockSpec((n_blk * t, d), lambda i: (i, 0)),
        out_shape=jax.ShapeDtypeStruct((n * t, d), jnp.float32),
    )(x, W, b_type)


def _final_call(partials, x, W_self, *, n_blk):
    n, d = x.shape

    def body(p_ref, x_ref, w_ref, out_ref):
        s = lax.dot(
            x_ref[...],
            w_ref[...],
            precision=lax.Precision.HIGHEST,
            preferred_element_type=jnp.float32,
        )
        out_ref[...] = jnp.maximum(p_ref[0] + p_ref[1] + s, 0.0)

    return pl.pallas_call(
        body,
        grid=(n // n_blk,),
        in_specs=[
            pl.BlockSpec((2, n_blk, d), lambda i: (0, i, 0)),
            pl.BlockSpec((n_blk, d), lambda i: (i, 0)),
            pl.BlockSpec((d, d), lambda i: (0, 0)),
        ],
        out_specs=pl.BlockSpec((n_blk, d), lambda i: (i, 0)),
        out_shape=jax.ShapeDtypeStruct((n, d), jnp.float32),
    )(partials, x, W_self)


def _sc_aggregate(htab2, gi4, di4, *, n, n_acc, d, n_ch):
    ch = _CH
    npt = (n // _NS) // 8 * 8
    wrem = n - _NS * npt
    zrem = n_acc - _NS * npt
    zch = 104
    assert npt % zch == 0 and wrem % 8 == 0 and zrem % 8 == 0
    assert max(wrem, zrem) <= ch and zch <= ch
    assert n_ch % 4 == 0 and n_ch >= 8
    mesh = plsc.VectorSubcoreMesh(
        core_axis_name="c", subcore_axis_name="s", num_cores=_NC, num_subcores=_NS
    )

    @functools.partial(
        pl.kernel,
        mesh=mesh,
        out_type=jax.ShapeDtypeStruct((_NC, n, d), jnp.float32),
        scratch_types=[
            pltpu.VMEM((4, 1, ch), jnp.int32),
            pltpu.VMEM((4, 1, ch), jnp.int32),
            pltpu.VMEM((2, ch, d), jnp.float32),
            pltpu.VMEM((zch, d), jnp.float32),
            pltpu.VMEM_SHARED((n_acc, d), jnp.float32),
            pltpu.SemaphoreType.DMA((4,)),
            pltpu.SemaphoreType.DMA((4,)),
            pltpu.SemaphoreType.DMA((2,)),
        ],
    )
    def run(tab_hbm, gi_hbm, di_hbm, out_hbm,
            gi_v, di_v, rows_v, zero_v, acc_sh, gsems, dsems, rsems):
        cid = lax.axis_index("c")
        sid = lax.axis_index("s")
        wid = cid * _NS + sid

        def zero_row(i, carry):
            for c in range(d // 16):
                zero_v[i, pl.ds(c * 16, 16)] = jnp.zeros((16,), jnp.float32)
            return carry

        lax.fori_loop(0, zch, zero_row, 0)
        for k in range(npt // zch):
            pltpu.sync_copy(zero_v, acc_sh.at[pl.ds(sid * npt + k * zch, zch)])

        @pl.when(sid == _NS - 1)
        def _zero_tail():
            pltpu.sync_copy(
                zero_v.at[pl.ds(0, zrem)], acc_sh.at[pl.ds(_NS * npt, zrem)]
            )

        plsc.subcore_barrier()

        def start_idx(j, s):
            pltpu.async_copy(gi_hbm.at[wid, j], gi_v.at[s], gsems.at[s])
            pltpu.async_copy(di_hbm.at[wid, j], di_v.at[s], dsems.at[s])

        def wait_idx(j, s):
            pltpu.make_async_copy(
                gi_hbm.at[wid, j], gi_v.at[s], gsems.at[s]
            ).wait()
            pltpu.make_async_copy(
                di_hbm.at[wid, j], di_v.at[s], dsems.at[s]
            ).wait()

        def start_gather(s, r):
            pltpu.async_copy(
                tab_hbm.at[gi_v.at[s, 0]], rows_v.at[r], rsems.at[r]
            )

        def wait_gather(s, r):
            pltpu.make_async_copy(
                tab_hbm.at[gi_v.at[s, 0]], rows_v.at[r], rsems.at[r]
            ).wait()

        def scatter(s, r):
            pltpu.sync_copy(
                rows_v.at[r], acc_sh.at[di_v.at[s, 0]], add=True
            )

        def step(j, jj):
            wait_idx(jj + 1, (j + 1) % 4)
            start_gather((j + 1) % 4, (j + 1) % 2)
            wait_gather(j % 4, j % 2)
            scatter(j % 4, j % 2)
            start_idx(jj + 3, (j + 3) % 4)

        start_idx(0, 0)
        start_idx(1, 1)
        start_idx(2, 2)
        wait_idx(0, 0)
        start_gather(0, 0)

        def quad(k, carry):
            j4 = 4 * k
            for u in range(4):
                step(u, j4 + u)
            return carry

        lax.fori_loop(0, (n_ch - 4) // 4, quad, 0)

        base = n_ch - 4
        for u in range(4):
            j = base + u
            if u < 3:
                wait_idx(j + 1, (j + 1) % 4)
                start_gather((j + 1) % 4, (j + 1) % 2)
            wait_gather(j % 4, j % 2)
            scatter(j % 4, j % 2)
            if u == 0:
                start_idx(n_ch - 1, (n_ch - 1) % 4)

        plsc.subcore_barrier()

        pltpu.sync_copy(
            acc_sh.at[pl.ds(sid * npt, npt)],
            out_hbm.at[cid, pl.ds(sid * npt, npt)],
        )

        @pl.when(sid == _NS - 1)
        def _write_tail():
            pltpu.sync_copy(
                acc_sh.at[pl.ds(_NS * npt, wrem)],
                out_hbm.at[cid, pl.ds(_NS * npt, wrem)],
            )

    return run(htab2, gi4, di4)


def kernel(x, edge_index, edge_type, W, W_self, b_type):
    n, d = x.shape
    e = edge_index.shape[1]
    t = b_type.shape[0]
    assert e % _NW == 0
    epw = e // _NW
    n_ch = -(-epw // _CH)
    pad = n_ch * _CH - epw
    n_acc = -(-(n + 1) // 8) * 8

    src = edge_index[0]
    dst = edge_index[1]
    gidx = src * t + edge_type

    def chunked(idx, fill):
        idx = idx.reshape(_NW, epw)
        if pad:
            filler = jnp.full((_NW, pad), fill, dtype=jnp.int32)
            idx = jnp.concatenate([idx, filler], axis=1)
        return idx.reshape(_NW, n_ch, 1, _CH)

    htab = _htab_call(x, W, b_type, n_blk=2000)
    partials = _sc_aggregate(
        htab,
        chunked(gidx, 0),
        chunked(dst, n),
        n=n, n_acc=n_acc, d=d, n_ch=n_ch,
    )
    return _final_call(partials, x, W_self, n_blk=2000)
